# Initial kernel scaffold; baseline (speedup 1.0000x reference)
import jax
import jax.numpy as jnp
from jax.experimental import pallas as pl


def kernel(x, edge_index, W, b, gamma, beta):
    raise NotImplementedError("write your pallas kernel here")


# speedup vs baseline: 5.7047x; 5.7047x over previous
import functools

import jax
import jax.numpy as jnp
from jax import lax
from jax.experimental import pallas as pl
from jax.experimental.pallas import tpu as pltpu
from jax.experimental.pallas import tpu_sc as plsc

N_NODES = 10000
N_EDGES = 320000
D = 128
BN_EPS = 1e-5

NC = 2
NS = 16
NW = NC * NS
CH = 128
STEPS = 79
E_PER_W = STEPS * CH
E_PAD = NW * E_PER_W
R_PAD = 10240
RPW = R_PAD // NS


def _sc_body(x_hbm, sidx_hbm, didx_hbm, zrow_hbm, zcnt_hbm,
             part_hbm, cnt_hbm,
             sidx_v, didx_v, rows_v, ones_v, acc_s, cnt_s, sem):
    cid = lax.axis_index("c")
    sid = lax.axis_index("s")
    wid = cid * NS + sid

    pltpu.sync_copy(zrow_hbm, acc_s.at[pl.ds(sid * RPW, RPW)])
    pltpu.sync_copy(zcnt_hbm, cnt_s.at[pl.ds(sid * RPW, RPW)])

    for i in range(CH // 16):
        ones_v[pl.ds(i * 16, 16)] = jnp.full((16,), 1.0, jnp.float32)

    pltpu.sync_copy(sidx_hbm.at[wid], sidx_v)
    pltpu.sync_copy(didx_hbm.at[wid], didx_v)

    plsc.subcore_barrier()

    def step(j, carry):
        pltpu.async_copy(x_hbm.at[sidx_v.at[j]], rows_v, sem).wait()
        pltpu.sync_copy(rows_v, acc_s.at[didx_v.at[j]], add=True)
        pltpu.sync_copy(ones_v, cnt_s.at[didx_v.at[j]], add=True)
        return carry

    lax.fori_loop(0, STEPS, step, 0)

    plsc.subcore_barrier()

    pltpu.sync_copy(acc_s.at[pl.ds(sid * RPW, RPW)],
                    part_hbm.at[cid, pl.ds(sid * RPW, RPW)])
    pltpu.sync_copy(cnt_s.at[pl.ds(sid * RPW, RPW)],
                    cnt_hbm.at[cid, pl.ds(sid * RPW, RPW)])


_sc_agg = pl.kernel(
    _sc_body,
    out_type=[
        jax.ShapeDtypeStruct((NC, R_PAD, D), jnp.float32),
        jax.ShapeDtypeStruct((NC, R_PAD), jnp.float32),
    ],
    mesh=plsc.VectorSubcoreMesh(core_axis_name="c", subcore_axis_name="s"),
    scratch_types=[
        pltpu.VMEM((STEPS, CH), jnp.int32),
        pltpu.VMEM((STEPS, CH), jnp.int32),
        pltpu.VMEM((CH, D), jnp.float32),
        pltpu.VMEM((CH,), jnp.float32),
        pltpu.VMEM_SHARED((R_PAD, D), jnp.float32),
        pltpu.VMEM_SHARED((R_PAD,), jnp.float32),
        pltpu.SemaphoreType.DMA,
    ],
)

BM = 1000


def _tc_a_body(part_ref, inv_ref, has_ref, w_ref, b_ref, pre_ref, stat_ref):
    i = pl.program_id(0)
    agg = part_ref[0] + part_ref[1]
    scaled = agg * inv_ref[...]
    pre = lax.dot_general(scaled, w_ref[...], (((1,), (1,)), ((), ())),
                          preferred_element_type=jnp.float32)
    pre = pre + has_ref[...] * b_ref[...]
    pre_ref[...] = pre

    @pl.when(i == 0)
    def _():
        stat_ref[...] = jnp.zeros_like(stat_ref)

    stat_ref[0:1] += jnp.sum(pre, axis=0, keepdims=True)
    stat_ref[1:2] += jnp.sum(pre * pre, axis=0, keepdims=True)


_tc_a = pl.pallas_call(
    _tc_a_body,
    grid=(N_NODES // BM,),
    in_specs=[
        pl.BlockSpec((2, BM, D), lambda i: (0, i, 0)),
        pl.BlockSpec((BM, 1), lambda i: (i, 0)),
        pl.BlockSpec((BM, 1), lambda i: (i, 0)),
        pl.BlockSpec((D, D), lambda i: (0, 0)),
        pl.BlockSpec((1, D), lambda i: (0, 0)),
    ],
    out_specs=[
        pl.BlockSpec((BM, D), lambda i: (i, 0)),
        pl.BlockSpec((2, D), lambda i: (0, 0)),
    ],
    out_shape=[
        jax.ShapeDtypeStruct((N_NODES, D), jnp.float32),
        jax.ShapeDtypeStruct((2, D), jnp.float32),
    ],
)


def _tc_b_body(pre_ref, stat_ref, g_ref, bt_ref, out_ref):
    inv_n = 1.0 / N_NODES
    mu = stat_ref[0:1] * inv_n
    var = stat_ref[1:2] * inv_n - mu * mu
    scale = g_ref[...] * lax.rsqrt(var + BN_EPS)
    out_ref[...] = jnp.maximum((pre_ref[...] - mu) * scale + bt_ref[...], 0.0)


_tc_b = pl.pallas_call(
    _tc_b_body,
    grid=(N_NODES // BM,),
    in_specs=[
        pl.BlockSpec((BM, D), lambda i: (i, 0)),
        pl.BlockSpec((2, D), lambda i: (0, 0)),
        pl.BlockSpec((1, D), lambda i: (0, 0)),
        pl.BlockSpec((1, D), lambda i: (0, 0)),
    ],
    out_specs=pl.BlockSpec((BM, D), lambda i: (i, 0)),
    out_shape=jax.ShapeDtypeStruct((N_NODES, D), jnp.float32),
)


def kernel(x, edge_index, W, b, gamma, beta):
    ei = edge_index.astype(jnp.int32)
    pad = E_PAD - N_EDGES
    src = jnp.concatenate([ei[0], jnp.zeros((pad,), jnp.int32)])
    dst = jnp.concatenate([ei[1], jnp.full((pad,), R_PAD - 1, jnp.int32)])
    src = src.reshape(NW, STEPS, CH)
    dst = dst.reshape(NW, STEPS, CH)
    zrow = jnp.zeros((RPW, D), jnp.float32)
    zcnt = jnp.zeros((RPW,), jnp.float32)

    part, cnt = _sc_agg(x, src, dst, zrow, zcnt)

    c = cnt[0] + cnt[1]
    inv = (1.0 / jnp.clip(c, 1.0, None))[:, None]
    has = (c > 0).astype(jnp.float32)[:, None]

    pre, stat = _tc_a(part, inv, has, W, b.reshape(1, D))
    return _tc_b(pre, stat, gamma.reshape(1, D), beta.reshape(1, D))
